# baseline (speedup 1.0000x reference)
import functools

import jax
import jax.numpy as jnp
import numpy as np
from jax import lax
from jax.experimental import pallas as pl
from jax.experimental.pallas import tpu as pltpu
from jax.experimental.pallas import tpu_sc as plsc

_COUNTS = [15, 8, 8]
B, C, N = 32, 256, 64
NW = 32

_L0 = set(range(15))
_L1 = set(range(16, 31, 2))
_L2 = set(range(34, 63, 4))
_O_CLASS = [_L0 | _L1 | _L2, _L0, _L0 | _L1, _L0]
_DEPTH_FULL = (62, 14, 30, 14)
_DEPTH_CAP = (30, 14, 30, 14)


def _static_mask() -> np.ndarray:
    mask = np.eye(N, dtype=bool)
    stride, offset = 1, 0
    for _, count in enumerate(_COUNTS):
        for _ in range(count):
            ii = np.arange(0, N - offset, stride)
            mask[ii, ii + offset] = True
            offset += stride
        offset += stride
        stride *= 2
    return mask


_MASK2D = _static_mask()

_mesh = plsc.VectorSubcoreMesh(
    core_axis_name="c", subcore_axis_name="s", num_cores=2, num_subcores=16
)


@functools.partial(
    pl.kernel,
    out_type=jax.ShapeDtypeStruct((B, N, 8, 2, 8, 128), jnp.float32),
    mesh=_mesh,
    compiler_params=pltpu.CompilerParams(
        needs_layout_passes=False, use_tc_tiling_on_sc=True
    ),
    scratch_types=[
        pltpu.VMEM((12, 2, 8, 128), jnp.float32),
        pltpu.VMEM((9, 2, 8, 128), jnp.float32),
        pltpu.VMEM((9, 2, 8, 128), jnp.float32),
        pltpu.VMEM((9, 2, 8, 128), jnp.float32),
        pltpu.VMEM((9, 2, 8, 128), jnp.float32),
        pltpu.SemaphoreType.DMA,
        pltpu.SemaphoreType.DMA,
        pltpu.SemaphoreType.DMA,
        pltpu.SemaphoreType.DMA,
    ],
)
def _sc_kernel(x_hbm, out_hbm, in_buf, ob0, ob1, ob2, ob3, s0, s1, s2, s3):
    wid = lax.axis_index("s") * 2 + lax.axis_index("c")
    obs = (ob0, ob1, ob2, ob3)
    sems = (s0, s1, s2, s3)

    pltpu.sync_copy(x_hbm.at[wid], in_buf.at[pl.ds(0, 8)])

    zeros16 = jnp.zeros((16,), jnp.float32)

    def zero_body(z, _):
        j8 = z // 8
        jr = z % 8
        for ob in obs:
            for ct in range(2):
                for m in range(8):
                    ob[j8, ct, jr, pl.ds(16 * m, 16)] = zeros16
        return 0

    lax.fori_loop(0, 64, zero_body, 0)

    def quad_block(o, r, depths, store_sets):
        rows = [r, r + 1, r + 2, r + 3]

        def gbody(g, _):
            ct = g // 2
            mco = (g % 2) * 64
            for rr in rows:
                ob = obs[rr % 4]
                for t in range(4):
                    e = rr - 4 + t
                    j8z = jnp.maximum(o + (e // 8), 0)
                    jrz = e % 8
                    for u in range(4):
                        ob[j8z, ct, jrz, pl.ds(mco + 16 * u, 16)] = zeros16
            v = {
                rr: [in_buf[o, ct, rr, pl.ds(mco + 16 * u, 16)] for u in range(4)]
                for rr in rows
            }
            for rr in rows:
                ob = obs[rr % 4]
                for u in range(4):
                    ob[o, ct, rr, pl.ds(mco + 16 * u, 16)] = v[rr][u]
            maxd = max(depths[rr % 4] for rr in rows)
            for d in range(1, maxd + 1):
                for rr in rows:
                    if d <= depths[rr % 4]:
                        q8, r8 = (rr + d) // 8, (rr + d) % 8
                        t8 = o + q8
                        v[rr] = [
                            jnp.maximum(
                                v[rr][u],
                                in_buf[t8, ct, r8, pl.ds(mco + 16 * u, 16)],
                            )
                            for u in range(4)
                        ]
                for rr in rows:
                    if d <= depths[rr % 4] and d in store_sets[rr % 4]:
                        ob = obs[rr % 4]
                        j8 = jnp.minimum(o + (rr + d) // 8, 8)
                        for u in range(4):
                            ob[j8, ct, (rr + d) % 8, pl.ds(mco + 16 * u, 16)] = v[
                                rr
                            ][u]
            return 0

        lax.fori_loop(0, 4, gbody, 0)

    def start(i, ob, sem):
        pltpu.async_copy(ob.at[pl.ds(0, 8)], out_hbm.at[wid, i], sem)

    def wait(i, ob, sem):
        pltpu.make_async_copy(ob.at[pl.ds(0, 8)], out_hbm.at[wid, i], sem).wait()

    def make_octet_body(depths, guard_first):
        store_sets = [
            sorted(d for d in _O_CLASS[c] if 1 <= d <= depths[c]) for c in range(4)
        ]

        def body(o, _):
            for r in (0, 4):
                for rr in range(r, r + 4):
                    cls = rr % 4
                    if guard_first and rr < 4:
                        @pl.when(o > 0)
                        def _(rr=rr, cls=cls):
                            wait(8 * o + rr - 4, obs[cls], sems[cls])
                    else:
                        wait(8 * o + rr - 4, obs[cls], sems[cls])
                quad_block(o, r, depths, store_sets)
                for rr in range(r, r + 4):
                    cls = rr % 4
                    start(8 * o + rr, obs[cls], sems[cls])
            return 0

        return body

    lax.fori_loop(0, 4, make_octet_body(_DEPTH_FULL, True), 0)
    lax.fori_loop(4, 8, make_octet_body(_DEPTH_CAP, False), 0)
    for r in range(4, 8):
        wait(56 + r, obs[r % 4], sems[r % 4])


@jax.jit
def kernel(x):
    x5 = jnp.transpose(x.reshape(B, 2, 128, 8, 8), (0, 3, 1, 4, 2))
    out = _sc_kernel(x5)
    x2d = jnp.transpose(out, (0, 3, 5, 1, 2, 4)).reshape(B, C, N, N)
    return x2d, jnp.asarray(_MASK2D)


# speedup vs baseline: 1.3492x; 1.3492x over previous
import functools

import jax
import jax.numpy as jnp
import numpy as np
from jax import lax
from jax.experimental import pallas as pl
from jax.experimental.pallas import tpu as pltpu
from jax.experimental.pallas import tpu_sc as plsc

_COUNTS = [15, 8, 8]
B, C, N = 32, 256, 64
NW = 32

_L0 = set(range(15))
_L1 = set(range(16, 31, 2))
_L2 = set(range(34, 63, 4))
_O_CLASS = [_L0 | _L1 | _L2, _L0, _L0 | _L1, _L0]
_DEPTH_FULL = (62, 14, 30, 14)
_DEPTH_CAP = (30, 14, 30, 14)


def _static_mask() -> np.ndarray:
    mask = np.eye(N, dtype=bool)
    stride, offset = 1, 0
    for _, count in enumerate(_COUNTS):
        for _ in range(count):
            ii = np.arange(0, N - offset, stride)
            mask[ii, ii + offset] = True
            offset += stride
        offset += stride
        stride *= 2
    return mask


_MASK2D = _static_mask()

_mesh = plsc.VectorSubcoreMesh(
    core_axis_name="c", subcore_axis_name="s", num_cores=2, num_subcores=16
)


@functools.partial(
    pl.kernel,
    out_type=jax.ShapeDtypeStruct((B, N, 8, 2, 8, 128), jnp.float32),
    mesh=_mesh,
    compiler_params=pltpu.CompilerParams(
        needs_layout_passes=False, use_tc_tiling_on_sc=True
    ),
    scratch_types=[
        pltpu.VMEM((12, 2, 8, 128), jnp.float32),
        pltpu.VMEM((9, 2, 8, 128), jnp.float32),
        pltpu.VMEM((9, 2, 8, 128), jnp.float32),
        pltpu.VMEM((9, 2, 8, 128), jnp.float32),
        pltpu.VMEM((9, 2, 8, 128), jnp.float32),
        pltpu.SemaphoreType.DMA,
        pltpu.SemaphoreType.DMA,
        pltpu.SemaphoreType.DMA,
        pltpu.SemaphoreType.DMA,
    ],
)
def _sc_kernel(x_hbm, out_hbm, in_buf, ob0, ob1, ob2, ob3, s0, s1, s2, s3):
    wid = lax.axis_index("s") * 2 + lax.axis_index("c")
    obs = (ob0, ob1, ob2, ob3)
    sems = (s0, s1, s2, s3)

    pltpu.sync_copy(x_hbm.at[wid], in_buf.at[pl.ds(0, 8)])

    zeros16 = jnp.zeros((16,), jnp.float32)

    def zero_body(z, _):
        j8 = z // 8
        jr = z % 8
        for ob in obs:
            for ct in range(2):
                for m in range(8):
                    ob[j8, ct, jr, pl.ds(16 * m, 16)] = zeros16
        return 0

    lax.fori_loop(0, 64, zero_body, 0)

    def pair_block(o, r, depth_a, ds_a, depth_b, ds_b, ob_a, ob_b):
        def gbody(g, _):
            ct = g // 2
            mco = (g % 2) * 64
            for rr, ob in ((r, ob_a), (r + 1, ob_b)):
                for t in range(4):
                    e = rr - 4 + t
                    j8z = jnp.maximum(o + (e // 8), 0)
                    jrz = e % 8
                    for u in range(4):
                        ob[j8z, ct, jrz, pl.ds(mco + 16 * u, 16)] = zeros16
            va = [in_buf[o, ct, r, pl.ds(mco + 16 * u, 16)] for u in range(4)]
            vb = [in_buf[o, ct, r + 1, pl.ds(mco + 16 * u, 16)] for u in range(4)]
            for u in range(4):
                ob_a[o, ct, r, pl.ds(mco + 16 * u, 16)] = va[u]
                ob_b[o, ct, r + 1, pl.ds(mco + 16 * u, 16)] = vb[u]
            for d in range(1, max(depth_a, depth_b) + 1):
                if d <= depth_a:
                    q8, r8 = (r + d) // 8, (r + d) % 8
                    t8 = o + q8
                    va = [
                        jnp.maximum(
                            va[u], in_buf[t8, ct, r8, pl.ds(mco + 16 * u, 16)]
                        )
                        for u in range(4)
                    ]
                if d <= depth_b:
                    q8b, r8b = (r + 1 + d) // 8, (r + 1 + d) % 8
                    t8b = o + q8b
                    vb = [
                        jnp.maximum(
                            vb[u], in_buf[t8b, ct, r8b, pl.ds(mco + 16 * u, 16)]
                        )
                        for u in range(4)
                    ]
                if d <= depth_a and d in ds_a:
                    j8 = jnp.minimum(o + (r + d) // 8, 8)
                    for u in range(4):
                        ob_a[j8, ct, (r + d) % 8, pl.ds(mco + 16 * u, 16)] = va[u]
                if d <= depth_b and d in ds_b:
                    j8b = jnp.minimum(o + (r + 1 + d) // 8, 8)
                    for u in range(4):
                        ob_b[j8b, ct, (r + 1 + d) % 8, pl.ds(mco + 16 * u, 16)] = vb[u]
            return 0

        lax.fori_loop(0, 4, gbody, 0)

    def start(i, ob, sem):
        pltpu.async_copy(ob.at[pl.ds(0, 8)], out_hbm.at[wid, i], sem)

    def wait(i, ob, sem):
        pltpu.make_async_copy(ob.at[pl.ds(0, 8)], out_hbm.at[wid, i], sem).wait()

    def make_octet_body(depths, guard_first):
        def body(o, _):
            for r in (0, 2, 4, 6):
                cls_a, cls_b = r % 4, (r + 1) % 4
                depth_a, depth_b = depths[cls_a], depths[cls_b]
                ds_a = sorted(d for d in _O_CLASS[cls_a] if 1 <= d <= depth_a)
                ds_b = sorted(d for d in _O_CLASS[cls_b] if 1 <= d <= depth_b)
                i = 8 * o + r
                for rr, cls in ((r, cls_a), (r + 1, cls_b)):
                    if guard_first and rr < 4:
                        @pl.when(o > 0)
                        def _(rr=rr, cls=cls):
                            wait(8 * o + rr - 4, obs[cls], sems[cls])
                    else:
                        wait(8 * o + rr - 4, obs[cls], sems[cls])
                pair_block(
                    o, r, depth_a, ds_a, depth_b, ds_b, obs[cls_a], obs[cls_b]
                )
                start(i, obs[cls_a], sems[cls_a])
                start(i + 1, obs[cls_b], sems[cls_b])
            return 0

        return body

    lax.fori_loop(0, 4, make_octet_body(_DEPTH_FULL, True), 0)
    lax.fori_loop(4, 8, make_octet_body(_DEPTH_CAP, False), 0)
    for r in range(4, 8):
        wait(56 + r, obs[r % 4], sems[r % 4])


@jax.jit
def kernel(x):
    x5 = jnp.transpose(x.reshape(B, 2, 128, 8, 8), (0, 3, 1, 4, 2))
    out = _sc_kernel(x5)
    x2d = jnp.transpose(out, (0, 3, 5, 1, 2, 4)).reshape(B, C, N, N)
    return x2d, jnp.asarray(_MASK2D)
